# Initial kernel scaffold; baseline (speedup 1.0000x reference)
import jax
import jax.numpy as jnp
from jax.experimental import pallas as pl


def kernel(x, layer0_cb_conv1_w, layer0_cb_conv1_b, layer0_blk0_conv1_w, layer0_blk0_conv1_b, layer0_blk0_conv2_w, layer0_blk0_conv2_b, layer0_blk1_conv1_w, layer0_blk1_conv1_b, layer0_blk1_conv2_w, layer0_blk1_conv2_b, layer1_cb_conv1_w, layer1_cb_conv1_b, layer1_blk0_conv1_w, layer1_blk0_conv1_b, layer1_blk0_conv2_w, layer1_blk0_conv2_b, layer1_blk1_conv1_w, layer1_blk1_conv1_b, layer1_blk1_conv2_w, layer1_blk1_conv2_b, layer2_cb_conv1_w, layer2_cb_conv1_b, layer2_blk0_conv1_w, layer2_blk0_conv1_b, layer2_blk0_conv2_w, layer2_blk0_conv2_b, layer2_blk1_conv1_w, layer2_blk1_conv1_b, layer2_blk1_conv2_w, layer2_blk1_conv2_b, layer3_cb_conv1_w, layer3_cb_conv1_b, layer3_blk0_conv1_w, layer3_blk0_conv1_b, layer3_blk0_conv2_w, layer3_blk0_conv2_b, layer3_blk1_conv1_w, layer3_blk1_conv1_b, layer3_blk1_conv2_w, layer3_blk1_conv2_b, fc_w, fc_b):
    raise NotImplementedError("write your pallas kernel here")


# speedup vs baseline: 5.1503x; 5.1503x over previous
import functools

import jax
import jax.numpy as jnp
from jax.experimental import pallas as pl
from jax.experimental.pallas import tpu as pltpu


GA = 64
GB = 64

_TAP = ((0, 0), (1, 0), (0, 1))


def _layer_kernel(ph_ref, cbw_ref, cbb_ref,
                  w1a_ref, b1a_ref, w2a_ref, b2a_ref,
                  w1b_ref, b1b_ref, w2b_ref, b2b_ref,
                  mask_ref, o_ref, sA, sB, sC,
                  *, B, Cin, Cout, Lp, Wp2, grouped):
    BL = B * Lp
    f32 = jnp.float32
    bf16 = jnp.bfloat16

    sA[:, :GA] = jnp.zeros((sA.shape[0], GA), bf16)
    for b in range(B):
        sA[:, GA + b * Lp:GA + (b + 1) * Lp] = ph_ref[b]

    mask = mask_ref[...]

    acc = None
    if grouped:
        for g in range(4):
            dr, dc = divmod(g, 2)
            s = (dr - 1) * Wp2 + (dc - 1)
            t = jnp.dot(cbw_ref[g], sA[:, GA + s:GA + s + BL],
                        preferred_element_type=f32)
            acc = t if acc is None else acc + t
    else:
        for kh in range(3):
            rp, dr = _TAP[kh]
            for kw in range(3):
                cp, dc = _TAP[kw]
                p = rp * 2 + cp
                s = (dr - 1) * Wp2 + (dc - 1)
                t = jnp.dot(cbw_ref[kh * 3 + kw],
                            sA[p * Cin:(p + 1) * Cin, GA + s:GA + s + BL],
                            preferred_element_type=f32)
                acc = t if acc is None else acc + t
    x0 = jnp.maximum(acc + cbb_ref[...], 0.0) * mask

    zg = jnp.zeros((Cout, GB), bf16)
    sB[:, :GB] = zg
    sB[:, GB + BL:] = zg
    sC[:, :GB] = zg
    sC[:, GB + BL:] = zg
    sB[:, GB:GB + BL] = x0.astype(bf16)

    def conv9(src, w_ref):
        a = None
        for kh in range(3):
            for kw in range(3):
                s = (kh - 1) * Wp2 + (kw - 1)
                t = jnp.dot(w_ref[kh * 3 + kw], src[:, GB + s:GB + s + BL],
                            preferred_element_type=f32)
                a = t if a is None else a + t
        return a

    h = jnp.maximum(conv9(sB, w1a_ref) + b1a_ref[...], 0.0) * mask
    sC[:, GB:GB + BL] = h.astype(bf16)
    xres = sB[:, GB:GB + BL].astype(f32)
    out0 = jnp.maximum(conv9(sC, w2a_ref) + b2a_ref[...] + xres, 0.0) * mask
    sB[:, GB:GB + BL] = out0.astype(bf16)

    h = jnp.maximum(conv9(sB, w1b_ref) + b1b_ref[...], 0.0) * mask
    sC[:, GB:GB + BL] = h.astype(bf16)
    xres = sB[:, GB:GB + BL].astype(f32)
    out1 = jnp.maximum(conv9(sC, w2b_ref) + b2b_ref[...] + xres, 0.0) * mask
    outb = out1.astype(bf16)
    for b in range(B):
        o_ref[b] = outb[:, b * Lp:(b + 1) * Lp]


def _phase_split(xp):
    N, C, Hp, Wp = xp.shape
    Hh, Wh = Hp // 2, Wp // 2
    p = xp.reshape(N, C, Hh, 2, Wh, 2).transpose(0, 3, 5, 1, 2, 4)
    p = jnp.pad(p, ((0, 0), (0, 0), (0, 0), (0, 0), (0, 1), (0, 1)))
    return p.reshape(N, 4 * C, (Hh + 1) * (Wh + 1)).astype(jnp.bfloat16)


def _grouped_cb_weights(w):
    Cout, Cin = w.shape[0], w.shape[1]
    zeros = jnp.zeros((Cout, Cin), w.dtype)
    groups = []
    for dr in (0, 1):
        for dc in (0, 1):
            cols = []
            for rp in (0, 1):
                for cp in (0, 1):
                    kh = {(0, 0): 0, (1, 0): 1, (0, 1): 2}.get((rp, dr))
                    kw = {(0, 0): 0, (1, 0): 1, (0, 1): 2}.get((cp, dc))
                    cols.append(zeros if kh is None or kw is None
                                else w[:, :, kh, kw])
            groups.append(jnp.concatenate(cols, axis=1))
    return jnp.stack(groups).astype(jnp.bfloat16)


def _tap_cb_weights(w):
    return jnp.transpose(w, (2, 3, 0, 1)).reshape(
        9, w.shape[0], w.shape[1]).astype(jnp.bfloat16)


def _blk_weights(w):
    C = w.shape[0]
    return jnp.transpose(w, (2, 3, 0, 1)).reshape(9, C, C).astype(jnp.bfloat16)


def _ring_mask(Hp2, Wp2, B):
    m = jnp.pad(jnp.ones((Hp2 - 2, Wp2 - 2), jnp.float32), ((1, 1), (1, 1)))
    return jnp.tile(m.reshape(1, Hp2 * Wp2), (1, B))


def _run_layer(ph, cbw, cbb, blk, *, B, Cin, Cout, Hp2, Wp2, grouped):
    N = ph.shape[0]
    Lp = Hp2 * Wp2
    BL = B * Lp
    Cin4 = ph.shape[1]
    mask = _ring_mask(Hp2, Wp2, B)
    (w1a, b1a, w2a, b2a), (w1b, b1b, w2b, b2b) = blk

    kern = functools.partial(_layer_kernel, B=B, Cin=Cin, Cout=Cout,
                             Lp=Lp, Wp2=Wp2, grouped=grouped)
    csts = lambda n: (0, 0, 0)
    cst2 = lambda n: (0, 0)
    wspec = pl.BlockSpec(cbw.shape, csts)
    bspec = pl.BlockSpec((Cout, 1), cst2)
    kspec = pl.BlockSpec((9, Cout, Cout), csts)
    out = pl.pallas_call(
        kern,
        out_shape=jax.ShapeDtypeStruct((N, Cout, Lp), jnp.bfloat16),
        grid=(N // B,),
        in_specs=[
            pl.BlockSpec((B, Cin4, Lp), lambda n: (n, 0, 0)),
            wspec, bspec,
            kspec, bspec, kspec, bspec,
            kspec, bspec, kspec, bspec,
            pl.BlockSpec((1, BL), cst2),
        ],
        out_specs=pl.BlockSpec((B, Cout, Lp), lambda n: (n, 0, 0)),
        scratch_shapes=[
            pltpu.VMEM((Cin4, GA + BL), jnp.bfloat16),
            pltpu.VMEM((Cout, GB + BL + GB), jnp.bfloat16),
            pltpu.VMEM((Cout, GB + BL + GB), jnp.bfloat16),
        ],
        compiler_params=pltpu.CompilerParams(
            dimension_semantics=("parallel",)),
    )(ph, cbw, cbb.reshape(Cout, 1).astype(jnp.float32),
      w1a, b1a.reshape(Cout, 1).astype(jnp.float32),
      w2a, b2a.reshape(Cout, 1).astype(jnp.float32),
      w1b, b1b.reshape(Cout, 1).astype(jnp.float32),
      w2b, b2b.reshape(Cout, 1).astype(jnp.float32),
      mask)
    return out


def kernel(x, layer0_cb_conv1_w, layer0_cb_conv1_b, layer0_blk0_conv1_w, layer0_blk0_conv1_b, layer0_blk0_conv2_w, layer0_blk0_conv2_b, layer0_blk1_conv1_w, layer0_blk1_conv1_b, layer0_blk1_conv2_w, layer0_blk1_conv2_b, layer1_cb_conv1_w, layer1_cb_conv1_b, layer1_blk0_conv1_w, layer1_blk0_conv1_b, layer1_blk0_conv2_w, layer1_blk0_conv2_b, layer1_blk1_conv1_w, layer1_blk1_conv1_b, layer1_blk1_conv2_w, layer1_blk1_conv2_b, layer2_cb_conv1_w, layer2_cb_conv1_b, layer2_blk0_conv1_w, layer2_blk0_conv1_b, layer2_blk0_conv2_w, layer2_blk0_conv2_b, layer2_blk1_conv1_w, layer2_blk1_conv1_b, layer2_blk1_conv2_w, layer2_blk1_conv2_b, layer3_cb_conv1_w, layer3_cb_conv1_b, layer3_blk0_conv1_w, layer3_blk0_conv1_b, layer3_blk0_conv2_w, layer3_blk0_conv2_b, layer3_blk1_conv1_w, layer3_blk1_conv1_b, layer3_blk1_conv2_w, layer3_blk1_conv2_b, fc_w, fc_b):
    cfgs = [
        dict(Cin=3,   Cout=64,  Hp2=58, Wp2=58, B=8, grouped=True),
        dict(Cin=64,  Cout=128, Hp2=30, Wp2=30, B=8, grouped=True),
        dict(Cin=128, Cout=256, Hp2=16, Wp2=16, B=8, grouped=True),
        dict(Cin=256, Cout=512, Hp2=9,  Wp2=9,  B=8, grouped=False),
    ]
    layer_params = [
        (layer0_cb_conv1_w, layer0_cb_conv1_b,
         ((layer0_blk0_conv1_w, layer0_blk0_conv1_b, layer0_blk0_conv2_w, layer0_blk0_conv2_b),
          (layer0_blk1_conv1_w, layer0_blk1_conv1_b, layer0_blk1_conv2_w, layer0_blk1_conv2_b))),
        (layer1_cb_conv1_w, layer1_cb_conv1_b,
         ((layer1_blk0_conv1_w, layer1_blk0_conv1_b, layer1_blk0_conv2_w, layer1_blk0_conv2_b),
          (layer1_blk1_conv1_w, layer1_blk1_conv1_b, layer1_blk1_conv2_w, layer1_blk1_conv2_b))),
        (layer2_cb_conv1_w, layer2_cb_conv1_b,
         ((layer2_blk0_conv1_w, layer2_blk0_conv1_b, layer2_blk0_conv2_w, layer2_blk0_conv2_b),
          (layer2_blk1_conv1_w, layer2_blk1_conv1_b, layer2_blk1_conv2_w, layer2_blk1_conv2_b))),
        (layer3_cb_conv1_w, layer3_cb_conv1_b,
         ((layer3_blk0_conv1_w, layer3_blk0_conv1_b, layer3_blk0_conv2_w, layer3_blk0_conv2_b),
          (layer3_blk1_conv1_w, layer3_blk1_conv1_b, layer3_blk1_conv2_w, layer3_blk1_conv2_b))),
    ]

    N = x.shape[0]
    xp = jnp.pad(x, ((0, 0), (0, 0), (1, 1), (1, 1)))
    for cfg, (cbw_raw, cbb, blk_raw) in zip(cfgs, layer_params):
        ph = _phase_split(xp)
        cbw = (_grouped_cb_weights(cbw_raw) if cfg["grouped"]
               else _tap_cb_weights(cbw_raw))
        blk = tuple((_blk_weights(w1), b1, _blk_weights(w2), b2)
                    for (w1, b1, w2, b2) in blk_raw)
        out = _run_layer(ph, cbw, cbb, blk, **cfg)
        xp = out.reshape(N, cfg["Cout"], cfg["Hp2"], cfg["Wp2"])

    x_final = xp[:, :, 1:-1, 1:-1].astype(jnp.float32)
    flat = x_final.reshape(N, -1)
    return flat @ fc_w + fc_b
